# baseline (speedup 1.0000x reference)
import jax
import jax.numpy as jnp
from jax import lax
from jax.experimental import pallas as pl
from jax.experimental.pallas import tpu as pltpu
from jax.experimental.pallas import tpu_sc as plsc

_NC = 2
_NS = 16
_NW = _NC * _NS
_CHUNK = 64
_BS = 1024


def _sc_body(table_hbm, out_hbm, buf):
    Bs, S, _ = out_hbm.shape
    rows_per_w = S // _NW
    wid = lax.axis_index("s") * _NC + lax.axis_index("c")
    base = wid * rows_per_w
    for c in range(rows_per_w // _CHUNK):
        r = base + c * _CHUNK
        pltpu.sync_copy(table_hbm.at[pl.ds(r, _CHUNK)], buf)
        for b in range(Bs):
            pltpu.sync_copy(buf, out_hbm.at[b, pl.ds(r, _CHUNK)])


def _tc_body(table_ref, out_ref):
    out_ref[...] = jnp.broadcast_to(table_ref[...][None, :, :], out_ref.shape)


def kernel(x, table):
    B, S, D = x.shape
    B_sc = 1
    B_tc = B - B_sc
    tc_out = pl.pallas_call(
        _tc_body,
        grid=(S // _BS,),
        in_specs=[pl.BlockSpec((_BS, D), lambda i: (i, 0))],
        out_specs=pl.BlockSpec((B_tc, _BS, D), lambda i: (0, i, 0)),
        out_shape=jax.ShapeDtypeStruct((B_tc, S, D), table.dtype),
    )(table)
    sc_out = pl.kernel(
        _sc_body,
        out_type=jax.ShapeDtypeStruct((B_sc, S, D), table.dtype),
        mesh=plsc.VectorSubcoreMesh(core_axis_name="c", subcore_axis_name="s"),
        scratch_types=[pltpu.VMEM((_CHUNK, D), table.dtype)],
    )(table)
    return jnp.concatenate([tc_out, sc_out], axis=0)


# speedup vs baseline: 2.1806x; 2.1806x over previous
import jax
import jax.numpy as jnp
from jax import lax
from jax.experimental import pallas as pl
from jax.experimental.pallas import tpu as pltpu
from jax.experimental.pallas import tpu_sc as plsc

_NC = 2
_NS = 16
_NW = _NC * _NS
_CHUNK = 32


def _sc_body(table_hbm, out_hbm, buf, rsem, wsem):
    B, S, _ = out_hbm.shape
    rows_per_w = S // _NW
    nchunks = rows_per_w // _CHUNK
    wid = lax.axis_index("s") * _NC + lax.axis_index("c")
    base = wid * rows_per_w

    def start_read(c):
        return pltpu.async_copy(
            table_hbm.at[pl.ds(base + c * _CHUNK, _CHUNK)],
            buf.at[c % 2], rsem)

    reads = [None] * nchunks
    writes = [None] * nchunks
    reads[0] = start_read(0)
    for c in range(nchunks):
        reads[c].wait()
        if c + 1 < nchunks:
            if c >= 1:
                for w in writes[c - 1]:
                    w.wait()
            reads[c + 1] = start_read(c + 1)
        r = base + c * _CHUNK
        writes[c] = [
            pltpu.async_copy(buf.at[c % 2], out_hbm.at[b, pl.ds(r, _CHUNK)], wsem)
            for b in range(B)
        ]
    for w in writes[nchunks - 2]:
        w.wait()
    for w in writes[nchunks - 1]:
        w.wait()


def kernel(x, table):
    B, S, D = x.shape
    f = pl.kernel(
        _sc_body,
        out_type=jax.ShapeDtypeStruct((B, S, D), table.dtype),
        mesh=plsc.VectorSubcoreMesh(core_axis_name="c", subcore_axis_name="s"),
        scratch_types=[
            pltpu.VMEM((2, _CHUNK, D), table.dtype),
            pltpu.SemaphoreType.DMA,
            pltpu.SemaphoreType.DMA,
        ],
    )
    return f(table)


# speedup vs baseline: 2.2668x; 1.0396x over previous
import jax
import jax.numpy as jnp
from jax import lax
from jax.experimental import pallas as pl
from jax.experimental.pallas import tpu as pltpu
from jax.experimental.pallas import tpu_sc as plsc

_NC = 2
_NS = 16
_NW = _NC * _NS
_CHUNK = 32
_NBUF = 3


def _sc_body(table_hbm, out_hbm, buf, rsem, wsem):
    B, S, _ = out_hbm.shape
    rows_per_w = S // _NW
    nchunks = rows_per_w // _CHUNK
    wid = lax.axis_index("s") * _NC + lax.axis_index("c")
    base = wid * rows_per_w

    def start_read(c):
        return pltpu.async_copy(
            table_hbm.at[pl.ds(base + c * _CHUNK, _CHUNK)],
            buf.at[c % _NBUF], rsem)

    reads = [None] * nchunks
    writes = [None] * nchunks
    for c in range(min(_NBUF - 1, nchunks)):
        reads[c] = start_read(c)
    for c in range(nchunks):
        reads[c].wait()
        if c + _NBUF - 1 < nchunks:
            if c >= 1:
                for w in writes[c - 1]:
                    w.wait()
            reads[c + _NBUF - 1] = start_read(c + _NBUF - 1)
        r = base + c * _CHUNK
        writes[c] = [
            pltpu.async_copy(
                buf.at[c % _NBUF],
                out_hbm.at[(b + c) % B, pl.ds(r, _CHUNK)], wsem)
            for b in range(B)
        ]
    for c in range(max(0, nchunks - _NBUF + 1), nchunks):
        for w in writes[c]:
            w.wait()


def kernel(x, table):
    B, S, D = x.shape
    f = pl.kernel(
        _sc_body,
        out_type=jax.ShapeDtypeStruct((B, S, D), table.dtype),
        mesh=plsc.VectorSubcoreMesh(core_axis_name="c", subcore_axis_name="s"),
        scratch_types=[
            pltpu.VMEM((_NBUF, _CHUNK, D), table.dtype),
            pltpu.SemaphoreType.DMA,
            pltpu.SemaphoreType.DMA,
        ],
    )
    return f(table)
